# baseline (speedup 1.0000x reference)
import functools

import jax
import jax.numpy as jnp
from jax import lax
from jax.experimental import pallas as pl
from jax.experimental.pallas import tpu as pltpu
from jax.experimental.pallas import tpu_sc as plsc

_NUM_ITEMS = 8
_L = 512
_C = 128
_CHUNK = 32
_NCHUNK = _C // _CHUNK
_NACC = _CHUNK // 16
_UNROLL = 2
_NSPLIT = 4


def _build():
    info = plsc.get_sparse_core_info()
    nc = info.num_cores
    mesh = plsc.VectorSubcoreMesh(core_axis_name="c", subcore_axis_name="s")

    @functools.partial(
        pl.kernel,
        out_type=tuple(
            jax.ShapeDtypeStruct((1, _C), jnp.float32)
            for _ in range(_NUM_ITEMS)
        ),
        mesh=mesh,
        scratch_types=[
            pltpu.VMEM((_NUM_ITEMS, 2), jnp.int32),
            pltpu.VMEM((_L, _CHUNK), jnp.float32),
            pltpu.VMEM((_CHUNK,), jnp.float32),
        ] + [pltpu.SemaphoreType.DMA] * _NSPLIT,
        compiler_params=pltpu.CompilerParams(
            use_tc_tiling_on_sc=False, needs_layout_passes=False,
            disable_bounds_checks=True, disable_semaphore_checks=True,
            skip_device_barrier=True),
    )
    def run(in_hbm, starts_hbm, *refs):
        outs = refs[:_NUM_ITEMS]
        slices_v, block_v, acc_v = refs[_NUM_ITEMS:_NUM_ITEMS + 3]
        sems = refs[_NUM_ITEMS + 3:]

        wid = lax.axis_index("s") * nc + lax.axis_index("c")
        item = wid // _NCHUNK
        c0 = (wid % _NCHUNK) * _CHUNK

        pltpu.sync_copy(starts_hbm, slices_v)
        g = plsc.load_gather(
            slices_v,
            [jnp.full((16,), item, jnp.int32), jnp.zeros((16,), jnp.int32)])
        row = jnp.max(g)

        cps = []
        for h in range(_NSPLIT):
            r0 = h * (_L // _NSPLIT)
            cps.append(pltpu.async_copy(
                in_hbm.at[row, pl.ds(r0, _L // _NSPLIT), pl.ds(c0, _CHUNK)],
                block_v.at[pl.ds(r0, _L // _NSPLIT)], sems[h]))

        zeros = jnp.zeros((16,), jnp.float32)

        def body(t, carry):
            accs = list(carry)
            r = t * _UNROLL
            for k in range(_UNROLL):
                for j in range(_NACC):
                    accs[j] = accs[j] + block_v[r + k, pl.ds(16 * j, 16)]
            return tuple(accs)

        accs = (zeros,) * _NACC
        per = _L // _NSPLIT // _UNROLL
        for h in range(_NSPLIT):
            cps[h].wait()
            accs = lax.fori_loop(h * per, (h + 1) * per, body, accs)
        for j in range(_NACC):
            acc_v[pl.ds(16 * j, 16)] = accs[j]

        for k in range(_NUM_ITEMS):
            @pl.when(item == k)
            def _(k=k):
                pltpu.sync_copy(acc_v, outs[k].at[0, pl.ds(c0, _CHUNK)])

    return run


_run = _build()


def kernel(input, slices):
    return tuple(
        o.reshape(1, _C, 1) for o in _run(input, slices.astype(jnp.int32)))


# speedup vs baseline: 1.0065x; 1.0065x over previous
import functools

import jax
import jax.numpy as jnp
from jax import lax
from jax.experimental import pallas as pl
from jax.experimental.pallas import tpu as pltpu
from jax.experimental.pallas import tpu_sc as plsc

_NUM_ITEMS = 8
_L = 512
_C = 128
_CHUNK = 64
_NCHUNK = _C // _CHUNK
_NACC = _CHUNK // 16
_UNROLL = 4
_NSPLIT = 4


def _build():
    mesh = plsc.VectorSubcoreMesh(
        core_axis_name="c", subcore_axis_name="s", num_cores=1)

    @functools.partial(
        pl.kernel,
        out_type=tuple(
            jax.ShapeDtypeStruct((1, _C), jnp.float32)
            for _ in range(_NUM_ITEMS)
        ),
        mesh=mesh,
        scratch_types=[
            pltpu.VMEM((_NUM_ITEMS, 2), jnp.int32),
            pltpu.VMEM((_L, _CHUNK), jnp.float32),
            pltpu.VMEM((_CHUNK,), jnp.float32),
        ] + [pltpu.SemaphoreType.DMA] * _NSPLIT,
        compiler_params=pltpu.CompilerParams(
            use_tc_tiling_on_sc=False, needs_layout_passes=False,
            disable_bounds_checks=True, disable_semaphore_checks=True,
            skip_device_barrier=True),
    )
    def run(in_hbm, starts_hbm, *refs):
        outs = refs[:_NUM_ITEMS]
        slices_v, block_v, acc_v = refs[_NUM_ITEMS:_NUM_ITEMS + 3]
        sems = refs[_NUM_ITEMS + 3:]

        wid = lax.axis_index("s")
        item = wid // _NCHUNK
        c0 = (wid % _NCHUNK) * _CHUNK

        pltpu.sync_copy(starts_hbm, slices_v)
        g = plsc.load_gather(
            slices_v,
            [jnp.full((16,), item, jnp.int32), jnp.zeros((16,), jnp.int32)])
        row = jnp.max(g)

        cps = []
        for h in range(_NSPLIT):
            r0 = h * (_L // _NSPLIT)
            cps.append(pltpu.async_copy(
                in_hbm.at[row, pl.ds(r0, _L // _NSPLIT), pl.ds(c0, _CHUNK)],
                block_v.at[pl.ds(r0, _L // _NSPLIT)], sems[h]))

        zeros = jnp.zeros((16,), jnp.float32)

        def body(t, carry):
            accs = list(carry)
            r = t * _UNROLL
            for k in range(_UNROLL):
                for j in range(_NACC):
                    accs[j] = accs[j] + block_v[r + k, pl.ds(16 * j, 16)]
            return tuple(accs)

        accs = (zeros,) * _NACC
        per = _L // _NSPLIT // _UNROLL
        for h in range(_NSPLIT):
            cps[h].wait()
            accs = lax.fori_loop(h * per, (h + 1) * per, body, accs)
        for j in range(_NACC):
            acc_v[pl.ds(16 * j, 16)] = accs[j]

        for k in range(_NUM_ITEMS):
            @pl.when(item == k)
            def _(k=k):
                pltpu.sync_copy(acc_v, outs[k].at[0, pl.ds(c0, _CHUNK)])

    return run


_run = _build()


def kernel(input, slices):
    return tuple(
        o.reshape(1, _C, 1) for o in _run(input, slices.astype(jnp.int32)))


# speedup vs baseline: 1.0151x; 1.0086x over previous
import functools

import jax
import jax.numpy as jnp
from jax import lax
from jax.experimental import pallas as pl
from jax.experimental.pallas import tpu as pltpu
from jax.experimental.pallas import tpu_sc as plsc

_NUM_ITEMS = 8
_L = 512
_C = 128
_CHUNK = 64
_NCHUNK = _C // _CHUNK
_NACC = _CHUNK // 16
_UNROLL = 2
_NSPLIT = 4


def _build():
    mesh = plsc.VectorSubcoreMesh(
        core_axis_name="c", subcore_axis_name="s", num_cores=1)

    @functools.partial(
        pl.kernel,
        out_type=tuple(
            jax.ShapeDtypeStruct((1, _C), jnp.float32)
            for _ in range(_NUM_ITEMS)
        ),
        mesh=mesh,
        scratch_types=[
            pltpu.VMEM((_NUM_ITEMS, 2), jnp.int32),
            pltpu.VMEM((_L, _CHUNK), jnp.float32),
            pltpu.VMEM((_CHUNK,), jnp.float32),
        ] + [pltpu.SemaphoreType.DMA] * _NSPLIT,
        compiler_params=pltpu.CompilerParams(
            use_tc_tiling_on_sc=False, needs_layout_passes=False,
            disable_bounds_checks=True, disable_semaphore_checks=True,
            skip_device_barrier=True),
    )
    def run(in_hbm, starts_hbm, *refs):
        outs = refs[:_NUM_ITEMS]
        slices_v, block_v, acc_v = refs[_NUM_ITEMS:_NUM_ITEMS + 3]
        sems = refs[_NUM_ITEMS + 3:]

        wid = lax.axis_index("s")
        item = wid // _NCHUNK
        c0 = (wid % _NCHUNK) * _CHUNK

        pltpu.sync_copy(starts_hbm, slices_v)
        g = plsc.load_gather(
            slices_v,
            [jnp.full((16,), item, jnp.int32), jnp.zeros((16,), jnp.int32)])
        row = jnp.max(g)

        cps = []
        for h in range(_NSPLIT):
            r0 = h * (_L // _NSPLIT)
            cps.append(pltpu.async_copy(
                in_hbm.at[row, pl.ds(r0, _L // _NSPLIT), pl.ds(c0, _CHUNK)],
                block_v.at[pl.ds(r0, _L // _NSPLIT)], sems[h]))

        zeros = jnp.zeros((16,), jnp.float32)

        def body(t, carry):
            accs = list(carry)
            r = t * _UNROLL
            for k in range(_UNROLL):
                for j in range(_NACC):
                    accs[j] = accs[j] + block_v[r + k, pl.ds(16 * j, 16)]
            return tuple(accs)

        accs = (zeros,) * _NACC
        per = _L // _NSPLIT // _UNROLL
        for h in range(_NSPLIT):
            cps[h].wait()
            accs = lax.fori_loop(h * per, (h + 1) * per, body, accs)
        for j in range(_NACC):
            acc_v[pl.ds(16 * j, 16)] = accs[j]

        for k in range(_NUM_ITEMS):
            @pl.when(item == k)
            def _(k=k):
                pltpu.sync_copy(acc_v, outs[k].at[0, pl.ds(c0, _CHUNK)])

    return run


_run = _build()


def kernel(input, slices):
    return tuple(
        o.reshape(1, _C, 1) for o in _run(input, slices.astype(jnp.int32)))


# speedup vs baseline: 1.0353x; 1.0199x over previous
import functools

import jax
import jax.numpy as jnp
from jax import lax
from jax.experimental import pallas as pl
from jax.experimental.pallas import tpu as pltpu
from jax.experimental.pallas import tpu_sc as plsc

_NUM_ITEMS = 8
_L = 512
_C = 128
_CHUNK = 64
_NCHUNK = _C // _CHUNK
_NACC = _CHUNK // 16
_UNROLL = 2
_NSPLIT = 2


def _build():
    mesh = plsc.VectorSubcoreMesh(
        core_axis_name="c", subcore_axis_name="s", num_cores=1)

    @functools.partial(
        pl.kernel,
        out_type=tuple(
            jax.ShapeDtypeStruct((1, _C), jnp.float32)
            for _ in range(_NUM_ITEMS)
        ),
        mesh=mesh,
        scratch_types=[
            pltpu.VMEM((_NUM_ITEMS, 2), jnp.int32),
            pltpu.VMEM((_L, _CHUNK), jnp.float32),
            pltpu.VMEM((_CHUNK,), jnp.float32),
        ] + [pltpu.SemaphoreType.DMA] * _NSPLIT,
        compiler_params=pltpu.CompilerParams(
            use_tc_tiling_on_sc=False, needs_layout_passes=False,
            disable_bounds_checks=True, disable_semaphore_checks=True,
            skip_device_barrier=True),
    )
    def run(in_hbm, starts_hbm, *refs):
        outs = refs[:_NUM_ITEMS]
        slices_v, block_v, acc_v = refs[_NUM_ITEMS:_NUM_ITEMS + 3]
        sems = refs[_NUM_ITEMS + 3:]

        wid = lax.axis_index("s")
        item = wid // _NCHUNK
        c0 = (wid % _NCHUNK) * _CHUNK

        pltpu.sync_copy(starts_hbm, slices_v)
        g = plsc.load_gather(
            slices_v,
            [jnp.full((16,), item, jnp.int32), jnp.zeros((16,), jnp.int32)])
        row = jnp.max(g)

        cps = []
        for h in range(_NSPLIT):
            r0 = h * (_L // _NSPLIT)
            cps.append(pltpu.async_copy(
                in_hbm.at[row, pl.ds(r0, _L // _NSPLIT), pl.ds(c0, _CHUNK)],
                block_v.at[pl.ds(r0, _L // _NSPLIT)], sems[h]))

        zeros = jnp.zeros((16,), jnp.float32)

        def body(t, carry):
            accs = list(carry)
            r = t * _UNROLL
            for k in range(_UNROLL):
                for j in range(_NACC):
                    accs[j] = accs[j] + block_v[r + k, pl.ds(16 * j, 16)]
            return tuple(accs)

        accs = (zeros,) * _NACC
        per = _L // _NSPLIT // _UNROLL
        for h in range(_NSPLIT):
            cps[h].wait()
            accs = lax.fori_loop(h * per, (h + 1) * per, body, accs)
        for j in range(_NACC):
            acc_v[pl.ds(16 * j, 16)] = accs[j]

        for k in range(_NUM_ITEMS):
            @pl.when(item == k)
            def _(k=k):
                pltpu.sync_copy(acc_v, outs[k].at[0, pl.ds(c0, _CHUNK)])

    return run


_run = _build()


def kernel(input, slices):
    return tuple(
        o.reshape(1, _C, 1) for o in _run(input, slices.astype(jnp.int32)))
